# baseline (speedup 1.0000x reference)
import functools

import jax
import jax.numpy as jnp
from jax import lax
from jax.experimental import pallas as pl
from jax.experimental.pallas import tpu as pltpu
from jax.experimental.pallas import tpu_sc as plsc

_N = 100000
_V = 119
_D = 128
_NW = 32
_CHUNK = 128
_SUB = 2
_SC_ROWS = _CHUNK * _SUB
_NSC = 13
_NBUF = 3
_PW = _SC_ROWS * _NSC
_LAST = _N - _PW


def _make_kernel():
    mesh = plsc.VectorSubcoreMesh(core_axis_name="c", subcore_axis_name="s")

    @functools.partial(
        pl.kernel,
        mesh=mesh,
        out_type=jax.ShapeDtypeStruct((_N, _D), jnp.float32),
        scratch_types=[
            pltpu.VMEM_SHARED((_V, _D), jnp.float32),
            pltpu.VMEM((_PW,), jnp.int32),
            pltpu.VMEM((_NBUF, _SC_ROWS, _D), jnp.float32),
            pltpu.SemaphoreType.DMA,
            pltpu.SemaphoreType.DMA,
            pltpu.SemaphoreType.DMA,
            pltpu.SemaphoreType.DMA,
            pltpu.SemaphoreType.DMA,
            pltpu.SemaphoreType.DMA,
        ],
    )
    def emb_kernel(z_hbm, table_hbm, out_hbm, table_sh, idx_v, rows_v,
                   sg0, sg1, sg2, sw0, sw1, sw2):
        wid = lax.axis_index("s") * 2 + lax.axis_index("c")
        base = ((wid * _LAST) // (_NW - 1)) // 8 * 8

        @pl.when(lax.axis_index("s") == 0)
        def _():
            pltpu.sync_copy(table_hbm, table_sh)

        pltpu.sync_copy(z_hbm.at[pl.ds(base, _PW)], idx_v)
        plsc.subcore_barrier()
        sg = (sg0, sg1, sg2)
        sw = (sw0, sw1, sw2)

        def gathers(c, b):
            for u in range(_SUB):
                yield pltpu.make_async_copy(
                    table_sh.at[idx_v.at[pl.ds((c * _SUB + u) * _CHUNK, _CHUNK)]],
                    rows_v.at[b, pl.ds(u * _CHUNK, _CHUNK)],
                    sg[b],
                )

        def write(c, b):
            return pltpu.make_async_copy(
                rows_v.at[b],
                out_hbm.at[pl.ds(base + c * _SC_ROWS, _SC_ROWS)],
                sw[b],
            )

        def chunk_step(c, b):
            for g in gathers(c, b):
                g.wait()
            write(c, b).start()

            @pl.when(c > 0)
            def _():
                write(c - 1, (b + 2) % _NBUF).wait()

            @pl.when(c + 2 < _NSC)
            def _():
                for g in gathers(c + 2, (b + 2) % _NBUF):
                    g.start()

        for g in gathers(0, 0):
            g.start()
        for g in gathers(1, 1):
            g.start()

        def body(i, carry):
            for b in (0, 1, 2):
                c = _NBUF * i + b

                @pl.when(c < _NSC)
                def _(c=c, b=b):
                    chunk_step(c, b)

            return carry

        lax.fori_loop(0, (_NSC + _NBUF - 1) // _NBUF, body, 0)
        write(_NSC - 1, (_NSC - 1) % _NBUF).wait()

    return emb_kernel


_emb = _make_kernel()


def kernel(Z, embeddings):
    return _emb(Z.astype(jnp.int32), embeddings)


# speedup vs baseline: 1.0946x; 1.0946x over previous
import functools

import jax
import jax.numpy as jnp
from jax import lax
from jax.experimental import pallas as pl
from jax.experimental.pallas import tpu as pltpu
from jax.experimental.pallas import tpu_sc as plsc

_N = 100000
_V = 119
_D = 128
_NW = 32
_CHUNK = 128
_NCH = 26
_NBUF = 6
_LOOKAHEAD = 3
_PW = _CHUNK * _NCH
_LAST = _N - _PW


def _make_kernel():
    mesh = plsc.VectorSubcoreMesh(core_axis_name="c", subcore_axis_name="s")

    @functools.partial(
        pl.kernel,
        mesh=mesh,
        out_type=jax.ShapeDtypeStruct((_N, _D), jnp.float32),
        scratch_types=[
            pltpu.VMEM_SHARED((_V, _D), jnp.float32),
            pltpu.VMEM((_PW,), jnp.int32),
            pltpu.VMEM((_NBUF, _CHUNK, _D), jnp.float32),
        ]
        + [pltpu.SemaphoreType.DMA] * (2 * _NBUF),
    )
    def emb_kernel(z_hbm, table_hbm, out_hbm, table_sh, idx_v, rows_v, *sems):
        sg = sems[:_NBUF]
        sw = sems[_NBUF:]
        wid = lax.axis_index("s") * 2 + lax.axis_index("c")
        base = ((wid * _LAST) // (_NW - 1)) // 8 * 8

        @pl.when(lax.axis_index("s") == 0)
        def _():
            pltpu.sync_copy(table_hbm, table_sh)

        pltpu.sync_copy(z_hbm.at[pl.ds(base, _PW)], idx_v)
        plsc.subcore_barrier()

        def gather(c, b):
            return pltpu.make_async_copy(
                table_sh.at[idx_v.at[pl.ds(c * _CHUNK, _CHUNK)]],
                rows_v.at[b],
                sg[b],
            )

        def write(c, b):
            return pltpu.make_async_copy(
                rows_v.at[b],
                out_hbm.at[pl.ds(base + c * _CHUNK, _CHUNK)],
                sw[b],
            )

        def chunk_step(c, b):
            gather(c, b).wait()
            write(c, b).start()

            @pl.when(c >= _LOOKAHEAD)
            def _():
                write(c - _LOOKAHEAD, (b - _LOOKAHEAD) % _NBUF).wait()

            @pl.when(c + _NBUF - _LOOKAHEAD < _NCH)
            def _():
                nxt = c + _NBUF - _LOOKAHEAD
                gather(nxt, (b + _NBUF - _LOOKAHEAD) % _NBUF).start()

        for c in range(_NBUF - _LOOKAHEAD):
            gather(c, c).start()

        def body(i, carry):
            for b in range(_NBUF):
                c = _NBUF * i + b

                @pl.when(c < _NCH)
                def _(c=c, b=b):
                    chunk_step(c, b)

            return carry

        lax.fori_loop(0, (_NCH + _NBUF - 1) // _NBUF, body, 0)
        for c in range(_NCH - _LOOKAHEAD, _NCH):
            write(c, c % _NBUF).wait()

    return emb_kernel


_emb = _make_kernel()


def kernel(Z, embeddings):
    return _emb(Z.astype(jnp.int32), embeddings)
